# baseline (speedup 1.0000x reference)
import functools

import jax
import jax.numpy as jnp
from jax import lax
from jax.experimental import pallas as pl
from jax.experimental.pallas import tpu as pltpu
from jax.experimental.pallas import tpu_sc as plsc

N = 10000
D = 128
K = 80
MM_BLK = 400
NBLK = 8
CH = 128
NW = 32
INT_MIN = -2147483648
INT_MAX = 2147483647


def _mm_body(x_ref, w_ref, o_ref):
    o_ref[...] = jnp.dot(x_ref[...], w_ref[...],
                         preferred_element_type=jnp.float32)


def _matmul(x, w):
    return pl.pallas_call(
        _mm_body,
        grid=(N // MM_BLK,),
        in_specs=[
            pl.BlockSpec((MM_BLK, D), lambda i: (i, 0)),
            pl.BlockSpec((D, D), lambda i: (0, 0)),
        ],
        out_specs=pl.BlockSpec((MM_BLK, D), lambda i: (i, 0)),
        out_shape=jax.ShapeDtypeStruct((N, D), jnp.float32),
    )(x, w)


def _sc_gather(table, idx):
    b = idx.shape[0]
    cpw = b // (NW * CH)
    mesh = plsc.VectorSubcoreMesh(core_axis_name="c", subcore_axis_name="s")

    @functools.partial(
        pl.kernel,
        out_type=jax.ShapeDtypeStruct((b, D), jnp.float32),
        mesh=mesh,
        scratch_types=[
            pltpu.VMEM((CH,), jnp.int32),
            pltpu.VMEM((CH, D), jnp.float32),
            pltpu.SemaphoreType.DMA,
        ],
    )
    def k(table_hbm, idx_hbm, out_hbm, idx_v, rows_v, sem):
        wid = lax.axis_index("s") * 2 + lax.axis_index("c")

        @pl.loop(0, cpw)
        def _chunk(i):
            base = (wid * cpw + i) * CH
            pltpu.sync_copy(idx_hbm.at[pl.ds(base, CH)], idx_v)
            pltpu.async_copy(table_hbm.at[idx_v], rows_v, sem).wait()
            pltpu.sync_copy(rows_v, out_hbm.at[pl.ds(base, CH)])

    return k(table, idx)


def _median_body(c_sref, m_sref, p_ref, b_ref, o_ref, *, relu):
    pid = pl.program_id(0)
    rows = lax.broadcasted_iota(jnp.int32, (K, D), 0)
    for j in range(NBLK):
        nid = pid * NBLK + j
        c = c_sref[nid]
        m1 = m_sref[nid] + 1
        seg = p_ref[j * K:(j + 1) * K, :]
        bits = lax.bitcast_convert_type(seg, jnp.int32)
        s = jnp.where(bits < 0, bits ^ jnp.int32(INT_MAX), bits)
        s = jnp.where(rows < c, s, jnp.int32(INT_MAX))
        cnt = jnp.sum((s < 0).astype(jnp.int32), axis=0, keepdims=True)
        base = jnp.where(cnt >= m1, jnp.int32(INT_MIN), jnp.int32(0))
        for bit in range(30, -1, -1):
            t = base + jnp.int32((1 << bit) - 1)
            cnt = jnp.sum((s <= t).astype(jnp.int32), axis=0, keepdims=True)
            base = jnp.where(cnt < m1, base + jnp.int32(1 << bit), base)
        mbits = jnp.where(base < 0, base ^ jnp.int32(INT_MAX), base)
        med = lax.bitcast_convert_type(mbits, jnp.float32)
        res = med + b_ref[0:1, :]
        if relu:
            res = jnp.maximum(res, 0.0)
        o_ref[j:j + 1, :] = res


def _median(p_flat, counts, med_off, bias, relu):
    grid_spec = pltpu.PrefetchScalarGridSpec(
        num_scalar_prefetch=2,
        grid=(N // NBLK,),
        in_specs=[
            pl.BlockSpec((NBLK * K, D), lambda i, c, m: (i, 0)),
            pl.BlockSpec((1, D), lambda i, c, m: (0, 0)),
        ],
        out_specs=pl.BlockSpec((NBLK, D), lambda i, c, m: (i, 0)),
    )
    return pl.pallas_call(
        functools.partial(_median_body, relu=relu),
        grid_spec=grid_spec,
        out_shape=jax.ShapeDtypeStruct((N, D), jnp.float32),
    )(counts, med_off, p_flat, bias.reshape(1, D))


def kernel(feat, edge_index, W1, b1, W2, b2):
    loop = jnp.arange(N, dtype=edge_index.dtype)
    src = jnp.concatenate([edge_index[0], loop])
    dst = jnp.concatenate([edge_index[1], loop])
    perm = jnp.argsort(dst)
    s_src = src[perm]
    counts = jnp.bincount(dst, length=N).astype(jnp.int32)
    starts = (jnp.cumsum(counts) - counts).astype(jnp.int32)
    med_off = (counts - 1) // 2
    k_iota = jnp.arange(K, dtype=jnp.int32)
    gidx = starts[:, None] + jnp.minimum(k_iota[None, :], counts[:, None] - 1)
    pidx = jnp.take(s_src, gidx.reshape(-1)).astype(jnp.int32)
    b_pad = ((N * K + NW * CH - 1) // (NW * CH)) * (NW * CH)
    pidx = jnp.concatenate(
        [pidx, jnp.zeros((b_pad - N * K,), jnp.int32)])

    h1 = _matmul(feat, W1)
    p1 = _sc_gather(h1, pidx)[:N * K]
    a1 = _median(p1, counts, med_off, b1, relu=True)
    h2 = _matmul(a1, W2)
    p2 = _sc_gather(h2, pidx)[:N * K]
    out = _median(p2, counts, med_off, b2, relu=False)
    return out


# speedup vs baseline: 5.2381x; 5.2381x over previous
import functools

import jax
import jax.numpy as jnp
from jax import lax
from jax.experimental import pallas as pl
from jax.experimental.pallas import tpu as pltpu
from jax.experimental.pallas import tpu_sc as plsc

N = 10000
D = 128
K = 80
MM_BLK = 400
NBLK = 80
CH = 128
NW = 32
INT_MIN = -2147483648
INT_MAX = 2147483647


def _mm_body(x_ref, w_ref, o_ref):
    o_ref[...] = jnp.dot(x_ref[...], w_ref[...],
                         preferred_element_type=jnp.float32)


def _matmul(x, w):
    return pl.pallas_call(
        _mm_body,
        grid=(N // MM_BLK,),
        in_specs=[
            pl.BlockSpec((MM_BLK, D), lambda i: (i, 0)),
            pl.BlockSpec((D, D), lambda i: (0, 0)),
        ],
        out_specs=pl.BlockSpec((MM_BLK, D), lambda i: (i, 0)),
        out_shape=jax.ShapeDtypeStruct((N, D), jnp.float32),
    )(x, w)


def _sc_gather(table, idx):
    b = idx.shape[0]
    cpw = b // (NW * CH)
    mesh = plsc.VectorSubcoreMesh(core_axis_name="c", subcore_axis_name="s")

    @functools.partial(
        pl.kernel,
        out_type=jax.ShapeDtypeStruct((b, D), jnp.float32),
        mesh=mesh,
        scratch_types=[
            pltpu.VMEM((CH,), jnp.int32),
            pltpu.VMEM((CH, D), jnp.float32),
            pltpu.SemaphoreType.DMA,
        ],
    )
    def k(table_hbm, idx_hbm, out_hbm, idx_v, rows_v, sem):
        wid = lax.axis_index("s") * 2 + lax.axis_index("c")

        @pl.loop(0, cpw)
        def _chunk(i):
            base = (wid * cpw + i) * CH
            pltpu.sync_copy(idx_hbm.at[pl.ds(base, CH)], idx_v)
            pltpu.async_copy(table_hbm.at[idx_v], rows_v, sem).wait()
            pltpu.sync_copy(rows_v, out_hbm.at[pl.ds(base, CH)])

    return k(table, idx)


def _median_body(c_sref, m_sref, p_ref, b_ref, o_ref, *, relu):
    pid = pl.program_id(0)
    rows = lax.broadcasted_iota(jnp.int32, (K, D), 0)
    for j in range(NBLK):
        nid = pid * NBLK + j
        c = c_sref[nid]
        m1 = m_sref[nid] + 1
        seg = p_ref[j * K:(j + 1) * K, :]
        bits = lax.bitcast_convert_type(seg, jnp.int32)
        s = jnp.where(bits < 0, bits ^ jnp.int32(INT_MAX), bits)
        s = jnp.where(rows < c, s, jnp.int32(INT_MAX))
        cnt = jnp.sum((s < 0).astype(jnp.int32), axis=0, keepdims=True)
        base = jnp.where(cnt >= m1, jnp.int32(INT_MIN), jnp.int32(0))
        for bit in range(30, -1, -1):
            t = base + jnp.int32((1 << bit) - 1)
            cnt = jnp.sum((s <= t).astype(jnp.int32), axis=0, keepdims=True)
            base = jnp.where(cnt < m1, base + jnp.int32(1 << bit), base)
        mbits = jnp.where(base < 0, base ^ jnp.int32(INT_MAX), base)
        med = lax.bitcast_convert_type(mbits, jnp.float32)
        res = med + b_ref[0:1, :]
        if relu:
            res = jnp.maximum(res, 0.0)
        o_ref[j:j + 1, :] = res


def _median(p_flat, counts, med_off, bias, relu):
    grid_spec = pltpu.PrefetchScalarGridSpec(
        num_scalar_prefetch=2,
        grid=(N // NBLK,),
        in_specs=[
            pl.BlockSpec((NBLK * K, D), lambda i, c, m: (i, 0)),
            pl.BlockSpec((1, D), lambda i, c, m: (0, 0)),
        ],
        out_specs=pl.BlockSpec((NBLK, D), lambda i, c, m: (i, 0)),
    )
    return pl.pallas_call(
        functools.partial(_median_body, relu=relu),
        grid_spec=grid_spec,
        out_shape=jax.ShapeDtypeStruct((N, D), jnp.float32),
    )(counts, med_off, p_flat, bias.reshape(1, D))


def kernel(feat, edge_index, W1, b1, W2, b2):
    loop = jnp.arange(N, dtype=edge_index.dtype)
    src = jnp.concatenate([edge_index[0], loop])
    dst = jnp.concatenate([edge_index[1], loop])
    perm = jnp.argsort(dst)
    s_src = src[perm]
    counts = jnp.bincount(dst, length=N).astype(jnp.int32)
    starts = (jnp.cumsum(counts) - counts).astype(jnp.int32)
    med_off = (counts - 1) // 2
    k_iota = jnp.arange(K, dtype=jnp.int32)
    gidx = (starts[:, None]
            + jnp.minimum(k_iota[None, :], counts[:, None] - 1)).reshape(-1)
    b_pad = ((N * K + NW * CH - 1) // (NW * CH)) * (NW * CH)
    gidx = jnp.concatenate(
        [gidx.astype(jnp.int32), jnp.zeros((b_pad - N * K,), jnp.int32)])
    e_pad = ((s_src.shape[0] + NW * CH - 1) // (NW * CH)) * (NW * CH)
    s_src_pad = jnp.concatenate(
        [s_src.astype(jnp.int32),
         jnp.zeros((e_pad - s_src.shape[0],), jnp.int32)])

    def layer(x, W, b, relu):
        h = _matmul(x, W)
        hs = _sc_gather(h, s_src_pad)
        p = _sc_gather(hs, gidx)[:N * K]
        return _median(p, counts, med_off, b, relu=relu)

    a1 = layer(feat, W1, b1, relu=True)
    return layer(a1, W2, b2, relu=False)


# speedup vs baseline: 5.7846x; 1.1043x over previous
import functools

import jax
import jax.numpy as jnp
from jax import lax
from jax.experimental import pallas as pl
from jax.experimental.pallas import tpu as pltpu
from jax.experimental.pallas import tpu_sc as plsc

N = 10000
D = 128
K = 80
MM_BLK = 400
NBLK = 80
CH = 128
NW = 32
INT_MIN = -2147483648
INT_MAX = 2147483647


def _mm_body(x_ref, w_ref, o_ref):
    o_ref[...] = jnp.dot(x_ref[...], w_ref[...],
                         preferred_element_type=jnp.float32)


def _matmul(x, w):
    return pl.pallas_call(
        _mm_body,
        grid=(N // MM_BLK,),
        in_specs=[
            pl.BlockSpec((MM_BLK, D), lambda i: (i, 0)),
            pl.BlockSpec((D, D), lambda i: (0, 0)),
        ],
        out_specs=pl.BlockSpec((MM_BLK, D), lambda i: (i, 0)),
        out_shape=jax.ShapeDtypeStruct((N, D), jnp.float32),
    )(x, w)


def _sc_gather(table, idx):
    b = idx.shape[0]
    cpw = b // (NW * CH)
    mesh = plsc.VectorSubcoreMesh(core_axis_name="c", subcore_axis_name="s")

    @functools.partial(
        pl.kernel,
        out_type=jax.ShapeDtypeStruct((b, D), jnp.float32),
        mesh=mesh,
        scratch_types=[
            pltpu.VMEM((2, CH), jnp.int32),
            pltpu.VMEM((2, CH, D), jnp.float32),
            pltpu.SemaphoreType.DMA,
            pltpu.SemaphoreType.DMA,
            pltpu.SemaphoreType.DMA,
            pltpu.SemaphoreType.DMA,
        ],
    )
    def k(table_hbm, idx_hbm, out_hbm, idx_v, rows_v, si0, si1, sg0, sg1):
        wid = lax.axis_index("s") * 2 + lax.axis_index("c")
        sis = (si0, si1)
        sgs = (sg0, sg1)

        @pl.loop(0, cpw, step=2)
        def _chunk(i):
            fetches = []
            for u in range(2):
                base = (wid * cpw + i + u) * CH
                fetches.append(pltpu.async_copy(
                    idx_hbm.at[pl.ds(base, CH)], idx_v.at[u], sis[u]))
            gathers = []
            for u in range(2):
                fetches[u].wait()
                gathers.append(pltpu.async_copy(
                    table_hbm.at[idx_v.at[u]], rows_v.at[u], sgs[u]))
            for u in range(2):
                base = (wid * cpw + i + u) * CH
                gathers[u].wait()
                pltpu.sync_copy(rows_v.at[u], out_hbm.at[pl.ds(base, CH)])

    return k(table, idx)


def _median_body(c_sref, m_sref, p_ref, b_ref, o_ref, *, relu):
    pid = pl.program_id(0)
    rows = lax.broadcasted_iota(jnp.int32, (K, D), 0)
    for j in range(NBLK):
        nid = pid * NBLK + j
        c = c_sref[nid]
        m1 = m_sref[nid] + 1
        seg = p_ref[j * K:(j + 1) * K, :]
        bits = lax.bitcast_convert_type(seg, jnp.int32)
        s = jnp.where(bits < 0, bits ^ jnp.int32(INT_MAX), bits)
        s = jnp.where(rows < c, s, jnp.int32(INT_MAX))
        cnt = jnp.sum((s < 0).astype(jnp.int32), axis=0, keepdims=True)
        base = jnp.where(cnt >= m1, jnp.int32(INT_MIN), jnp.int32(0))
        for bit in range(30, -1, -1):
            t = base + jnp.int32((1 << bit) - 1)
            cnt = jnp.sum((s <= t).astype(jnp.int32), axis=0, keepdims=True)
            base = jnp.where(cnt < m1, base + jnp.int32(1 << bit), base)
        mbits = jnp.where(base < 0, base ^ jnp.int32(INT_MAX), base)
        med = lax.bitcast_convert_type(mbits, jnp.float32)
        res = med + b_ref[0:1, :]
        if relu:
            res = jnp.maximum(res, 0.0)
        o_ref[j:j + 1, :] = res


def _median(p_flat, counts, med_off, bias, relu):
    grid_spec = pltpu.PrefetchScalarGridSpec(
        num_scalar_prefetch=2,
        grid=(N // NBLK,),
        in_specs=[
            pl.BlockSpec((NBLK * K, D), lambda i, c, m: (i, 0)),
            pl.BlockSpec((1, D), lambda i, c, m: (0, 0)),
        ],
        out_specs=pl.BlockSpec((NBLK, D), lambda i, c, m: (i, 0)),
    )
    return pl.pallas_call(
        functools.partial(_median_body, relu=relu),
        grid_spec=grid_spec,
        out_shape=jax.ShapeDtypeStruct((N, D), jnp.float32),
    )(counts, med_off, p_flat, bias.reshape(1, D))


def kernel(feat, edge_index, W1, b1, W2, b2):
    loop = jnp.arange(N, dtype=edge_index.dtype)
    src = jnp.concatenate([edge_index[0], loop])
    dst = jnp.concatenate([edge_index[1], loop])
    perm = jnp.argsort(dst)
    s_src = src[perm]
    counts = jnp.bincount(dst, length=N).astype(jnp.int32)
    starts = (jnp.cumsum(counts) - counts).astype(jnp.int32)
    med_off = (counts - 1) // 2
    k_iota = jnp.arange(K, dtype=jnp.int32)
    gidx = (starts[:, None]
            + jnp.minimum(k_iota[None, :], counts[:, None] - 1)).reshape(-1)
    unit = NW * CH * 2
    b_pad = ((N * K + unit - 1) // unit) * unit
    gidx = jnp.concatenate(
        [gidx.astype(jnp.int32), jnp.zeros((b_pad - N * K,), jnp.int32)])
    e_pad = ((s_src.shape[0] + unit - 1) // unit) * unit
    s_src_pad = jnp.concatenate(
        [s_src.astype(jnp.int32),
         jnp.zeros((e_pad - s_src.shape[0],), jnp.int32)])

    def layer(x, W, b, relu):
        h = _matmul(x, W)
        hs = _sc_gather(h, s_src_pad)
        p = _sc_gather(hs, gidx)
        return _median(p, counts, med_off, b, relu=relu)

    a1 = layer(feat, W1, b1, relu=True)
    return layer(a1, W2, b2, relu=False)
